# baseline (speedup 1.0000x reference)
import functools

import jax
import jax.numpy as jnp
from jax import lax
from jax.experimental import pallas as pl
from jax.experimental.pallas import tpu as pltpu
from jax.experimental.pallas import tpu_sc as plsc

B = 16384
F = 64
NC = 2
NS = 16
NW = NC * NS
BPW = B // NW
CHUNK = 128
NCHUNK = BPW // CHUNK
GROUPS = BPW // 16

_mesh = plsc.VectorSubcoreMesh(core_axis_name="c", subcore_axis_name="s")


@functools.partial(
    pl.kernel,
    mesh=_mesh,
    out_type=jax.ShapeDtypeStruct((NW, BPW), jnp.float32),
    scratch_types=[
        pltpu.VMEM((NCHUNK, CHUNK), jnp.int32),
        pltpu.VMEM((NCHUNK, CHUNK), jnp.int32),
        pltpu.VMEM((BPW, F), jnp.float32),
        pltpu.VMEM((BPW, F), jnp.float32),
        pltpu.VMEM((BPW,), jnp.float32),
        pltpu.SemaphoreType.DMA,
    ],
    compiler_params=pltpu.CompilerParams(
        needs_layout_passes=False, use_tc_tiling_on_sc=False),
)
def _cf_kernel(user_hbm, item_hbm, ut_hbm, it_hbm, out_hbm,
               uidx_v, iidx_v, urows_v, irows_v, out_v, sem):
    wid = lax.axis_index("s") * NC + lax.axis_index("c")

    pltpu.sync_copy(user_hbm.at[wid], uidx_v)
    pltpu.sync_copy(item_hbm.at[wid], iidx_v)

    copies = []
    for j in range(NCHUNK):
        copies.append(pltpu.async_copy(
            ut_hbm.at[uidx_v.at[j]], urows_v.at[pl.ds(j * CHUNK, CHUNK)], sem))
        copies.append(pltpu.async_copy(
            it_hbm.at[iidx_v.at[j]], irows_v.at[pl.ds(j * CHUNK, CHUNK)], sem))
    for cp in copies:
        cp.wait()

    lane = lax.iota(jnp.int32, 16)

    def group_body(g, carry):
        rows = g * 16 + lane
        acc = jnp.zeros((16,), jnp.float32)
        for c in range(F):
            col = jnp.full((16,), c, jnp.int32)
            u = plsc.load_gather(urows_v, [rows, col])
            iv = plsc.load_gather(irows_v, [rows, col])
            acc = acc + u * iv
        out_v[pl.ds(g * 16, 16)] = 1.0 / (1.0 + jnp.exp(-acc))
        return carry

    lax.fori_loop(0, GROUPS, group_body, 0)

    pltpu.sync_copy(out_v, out_hbm.at[wid])


def kernel(user, item, user_table, item_table):
    u2 = user.astype(jnp.int32).reshape(NW, NCHUNK, CHUNK)
    i2 = item.astype(jnp.int32).reshape(NW, NCHUNK, CHUNK)
    out = _cf_kernel(u2, i2, user_table, item_table)
    return out.reshape(B)


# speedup vs baseline: 2.3188x; 2.3188x over previous
import functools

import jax
import jax.numpy as jnp
from jax import lax
from jax.experimental import pallas as pl
from jax.experimental.pallas import tpu as pltpu
from jax.experimental.pallas import tpu_sc as plsc

B = 16384
F = 64
ROWS_PER_TILE = 8
N_TILES = 1000000 // ROWS_PER_TILE
NC = 2
NS = 16
NW = NC * NS
BPW = B // NW

_mesh = plsc.VectorSubcoreMesh(core_axis_name="c", subcore_axis_name="s")


@functools.partial(
    pl.kernel,
    mesh=_mesh,
    out_type=jax.ShapeDtypeStruct((NW, BPW), jnp.float32),
    scratch_types=[
        pltpu.VMEM((8, 64), jnp.int32),
        pltpu.VMEM((8, 64), jnp.int32),
        pltpu.VMEM((BPW // 2, 2 * F), jnp.float32),
        pltpu.VMEM((BPW // 2, 2 * F), jnp.float32),
        pltpu.VMEM((BPW,), jnp.float32),
        pltpu.SemaphoreType.DMA,
    ],
    compiler_params=pltpu.CompilerParams(needs_layout_passes=False),
)
def _cf_kernel(user_hbm, item_hbm, ut_hbm, it_hbm, out_hbm,
               uraw_v, iraw_v, urows_v, irows_v, out_v, sem):
    wid = lax.axis_index("s") * NC + lax.axis_index("c")

    pltpu.sync_copy(user_hbm.at[wid], uraw_v)
    pltpu.sync_copy(item_hbm.at[wid], iraw_v)

    def fire_body(g, carry):
        u16 = uraw_v[g >> 2, pl.ds((g & 3) * 16, 16)]
        i16 = iraw_v[g >> 2, pl.ds((g & 3) * 16, 16)]
        for l in range(16):
            uidx = u16[l]
            iidx = i16[l]
            dst_row = g * 8 + (l >> 1)
            dst_off = (l & 1) * F
            pltpu.async_copy(
                ut_hbm.at[uidx >> 3, uidx & 7],
                urows_v.at[dst_row, pl.ds(dst_off, F)], sem)
            pltpu.async_copy(
                it_hbm.at[iidx >> 3, iidx & 7],
                irows_v.at[dst_row, pl.ds(dst_off, F)], sem)
        return carry

    lax.fori_loop(0, BPW // 16, fire_body, 0)

    def drain_body(k, carry):
        pltpu.make_async_copy(
            ut_hbm.at[0, 0], urows_v.at[0, pl.ds(0, F)], sem).wait()
        pltpu.make_async_copy(
            it_hbm.at[0, 0], irows_v.at[0, pl.ds(0, F)], sem).wait()
        return carry

    lax.fori_loop(0, BPW, drain_body, 0)

    lane = lax.iota(jnp.int32, 16)
    half = lane >> 1
    colbase = (lane & 1) * F

    def group_body(g, carry):
        rvec = g * 8 + half
        acc = jnp.zeros((16,), jnp.float32)
        for c in range(F):
            cvec = colbase + c
            u = plsc.load_gather(urows_v, [rvec, cvec])
            iv = plsc.load_gather(irows_v, [rvec, cvec])
            acc = acc + u * iv
        out_v[pl.ds(g * 16, 16)] = 1.0 / (1.0 + jnp.exp(-acc))
        return carry

    lax.fori_loop(0, BPW // 16, group_body, 0)

    pltpu.sync_copy(out_v, out_hbm.at[wid])


def kernel(user, item, user_table, item_table):
    u2 = user.astype(jnp.int32).reshape(NW, 8, 64)
    i2 = item.astype(jnp.int32).reshape(NW, 8, 64)
    ut3 = user_table.reshape(N_TILES, ROWS_PER_TILE, F)
    it3 = item_table.reshape(N_TILES, ROWS_PER_TILE, F)
    out = _cf_kernel(u2, i2, ut3, it3)
    return out.reshape(B)
